# baseline (speedup 1.0000x reference)
import functools

import jax
import jax.numpy as jnp
from jax import lax
from jax.experimental import pallas as pl
from jax.experimental.pallas import tpu as pltpu
from jax.experimental.pallas import tpu_sc as plsc

MEMC = 1024
CHUNK = 4096
CB = 32
TOK_PER_W = 1024


def _idx_body(x_ref, mn_ref, o_ref):
    xc = x_ref[0]
    n = jnp.sqrt(jnp.sum(xc * xc, axis=0, keepdims=True))
    xn = xc / jnp.maximum(n, 1e-12)
    sim = jax.lax.dot_general(xn, mn_ref[...], (((0,), (1,)), ((), ())))
    o_ref[0, 0] = jnp.argmax(sim, axis=1).astype(jnp.int32)


def _make_sc_gather(B, C, HW):
    mesh = plsc.VectorSubcoreMesh(core_axis_name="c", subcore_axis_name="s")
    n_cblk = C // CB

    @functools.partial(
        pl.kernel,
        mesh=mesh,
        out_type=jax.ShapeDtypeStruct((B, C, HW), jnp.float32),
        scratch_types=[
            pltpu.VMEM((TOK_PER_W,), jnp.int32),
            pltpu.VMEM((CB * MEMC,), jnp.float32),
            pltpu.VMEM((CB, TOK_PER_W), jnp.float32),
        ],
        compiler_params=pltpu.CompilerParams(needs_layout_passes=False),
    )
    def sc_gather(memT_hbm, idx_hbm, out_hbm, idx_v, mt_v, ob_v):
        w = lax.axis_index("s") * 2 + lax.axis_index("c")
        b = w // 4
        q = w % 4
        base = b * HW + q * TOK_PER_W
        pltpu.sync_copy(idx_hbm.at[pl.ds(base, TOK_PER_W)], idx_v)
        for cb in range(n_cblk):
            pltpu.sync_copy(memT_hbm.at[pl.ds(cb * CB * MEMC, CB * MEMC)], mt_v)

            def body(tg, carry):
                iv = idx_v[pl.ds(tg * 16, 16)]
                for c in range(CB):
                    val = plsc.load_gather(mt_v, [iv + (c * MEMC)])
                    ob_v[c, pl.ds(tg * 16, 16)] = val
                return carry

            lax.fori_loop(0, TOK_PER_W // 16, body, 0)
            pltpu.sync_copy(
                ob_v,
                out_hbm.at[b, pl.ds(cb * CB, CB), pl.ds(q * TOK_PER_W, TOK_PER_W)],
            )

    return sc_gather


@jax.jit
def kernel(x, memory):
    B, C, H, W = x.shape
    HW = H * W
    x3 = x.reshape(B, C, HW)

    mn = jnp.linalg.norm(memory, axis=1, keepdims=True)
    mem_norm = memory / jnp.maximum(mn, 1e-12)

    NT = HW // CHUNK
    idx = pl.pallas_call(
        _idx_body,
        grid=(B, NT),
        in_specs=[
            pl.BlockSpec((1, C, CHUNK), lambda b, t: (b, 0, t)),
            pl.BlockSpec((MEMC, C), lambda b, t: (0, 0)),
        ],
        out_specs=pl.BlockSpec((1, 1, CHUNK), lambda b, t: (b * NT + t, 0, 0)),
        out_shape=jax.ShapeDtypeStruct((B * NT, 1, CHUNK), jnp.int32),
    )(x3, mem_norm)

    memT = memory.T.reshape(C * MEMC)
    out = _make_sc_gather(B, C, HW)(memT, idx.reshape(B * HW))
    return out.reshape(B, C, H, W)


# speedup vs baseline: 1.1345x; 1.1345x over previous
import functools

import jax
import jax.numpy as jnp
from jax import lax
from jax.experimental import pallas as pl
from jax.experimental.pallas import tpu as pltpu
from jax.experimental.pallas import tpu_sc as plsc

MEMC = 1024
CHUNK = 4096
CB = 16
TOK_PER_W = 1024


def _idx_body(x_ref, mn_ref, o_ref):
    xc = x_ref[0]
    n = jnp.sqrt(jnp.sum(xc * xc, axis=0, keepdims=True))
    xn = xc / jnp.maximum(n, 1e-12)
    sim = jax.lax.dot_general(xn, mn_ref[...], (((0,), (1,)), ((), ())))
    o_ref[0, 0] = jnp.argmax(sim, axis=1).astype(jnp.int32)


def _make_sc_gather(B, C, HW):
    mesh = plsc.VectorSubcoreMesh(core_axis_name="c", subcore_axis_name="s")
    n_cblk = C // CB

    @functools.partial(
        pl.kernel,
        mesh=mesh,
        out_type=jax.ShapeDtypeStruct((B, C, HW), jnp.float32),
        scratch_types=[
            pltpu.VMEM((TOK_PER_W,), jnp.int32),
            pltpu.VMEM((CB * MEMC,), jnp.float32),
            pltpu.VMEM((CB * MEMC,), jnp.float32),
            pltpu.VMEM((CB, TOK_PER_W), jnp.float32),
            pltpu.VMEM((CB, TOK_PER_W), jnp.float32),
            pltpu.SemaphoreType.DMA,
            pltpu.SemaphoreType.DMA,
            pltpu.SemaphoreType.DMA,
            pltpu.SemaphoreType.DMA,
        ],
        compiler_params=pltpu.CompilerParams(needs_layout_passes=False),
    )
    def sc_gather(memT_hbm, idx_hbm, out_hbm, idx_v, mt0, mt1, ob0, ob1, s0, s1, t0, t1):
        w = lax.axis_index("s") * 2 + lax.axis_index("c")
        b = w // 4
        q = w % 4
        base = b * HW + q * TOK_PER_W
        mt_bufs = (mt0, mt1)
        ob_bufs = (ob0, ob1)
        mt_sems = (s0, s1)
        ob_sems = (t0, t1)
        pltpu.sync_copy(idx_hbm.at[pl.ds(base, TOK_PER_W)], idx_v)
        loads = [None, None]
        stores = [None, None]
        loads[0] = pltpu.async_copy(
            memT_hbm.at[pl.ds(0, CB * MEMC)], mt_bufs[0], mt_sems[0]
        )
        for cb in range(n_cblk):
            cur = cb % 2
            if cb + 1 < n_cblk:
                loads[1 - cur] = pltpu.async_copy(
                    memT_hbm.at[pl.ds((cb + 1) * CB * MEMC, CB * MEMC)],
                    mt_bufs[1 - cur],
                    mt_sems[1 - cur],
                )
            loads[cur].wait()
            if stores[cur] is not None:
                stores[cur].wait()
            mt = mt_bufs[cur]
            ob = ob_bufs[cur]

            def body(tg, carry):
                iv = idx_v[pl.ds(tg * 16, 16)]
                for c in range(CB):
                    val = plsc.load_gather(mt, [iv + (c * MEMC)])
                    ob[c, pl.ds(tg * 16, 16)] = val
                return carry

            lax.fori_loop(0, TOK_PER_W // 16, body, 0)
            stores[cur] = pltpu.async_copy(
                ob,
                out_hbm.at[b, pl.ds(cb * CB, CB), pl.ds(q * TOK_PER_W, TOK_PER_W)],
                ob_sems[cur],
            )
        for st in stores:
            if st is not None:
                st.wait()

    return sc_gather


@jax.jit
def kernel(x, memory):
    B, C, H, W = x.shape
    HW = H * W
    x3 = x.reshape(B, C, HW)

    mn = jnp.linalg.norm(memory, axis=1, keepdims=True)
    mem_norm = memory / jnp.maximum(mn, 1e-12)

    NT = HW // CHUNK
    idx = pl.pallas_call(
        _idx_body,
        grid=(B, NT),
        in_specs=[
            pl.BlockSpec((1, C, CHUNK), lambda b, t: (b, 0, t)),
            pl.BlockSpec((MEMC, C), lambda b, t: (0, 0)),
        ],
        out_specs=pl.BlockSpec((1, 1, CHUNK), lambda b, t: (b * NT + t, 0, 0)),
        out_shape=jax.ShapeDtypeStruct((B * NT, 1, CHUNK), jnp.int32),
    )(x3, mem_norm)

    memT = memory.T.reshape(C * MEMC)
    out = _make_sc_gather(B, C, HW)(memT, idx.reshape(B * HW))
    return out.reshape(B, C, H, W)


# speedup vs baseline: 1.9817x; 1.7468x over previous
import functools

import jax
import jax.numpy as jnp
from jax.experimental import pallas as pl
from jax.experimental.pallas import tpu as pltpu

MEMC = 1024
CHUNK = 4096


def _vq_body(x_ref, mn_ref, mem_ref, o_ref):
    xc = x_ref[0]
    n = jnp.sqrt(jnp.sum(xc * xc, axis=0, keepdims=True))
    xn = xc / jnp.maximum(n, 1e-12)
    sim = jax.lax.dot_general(xn, mn_ref[...], (((0,), (1,)), ((), ())))
    idx = jnp.argmax(sim, axis=1)
    k_iota = jax.lax.broadcasted_iota(jnp.int32, (CHUNK, MEMC), 1)
    oh = (k_iota == idx[:, None]).astype(jnp.float32)
    o_ref[0] = jax.lax.dot_general(mem_ref[...], oh, (((0,), (1,)), ((), ())))


@jax.jit
def kernel(x, memory):
    B, C, H, W = x.shape
    HW = H * W
    x3 = x.reshape(B, C, HW)

    mn = jnp.linalg.norm(memory, axis=1, keepdims=True)
    mem_norm = memory / jnp.maximum(mn, 1e-12)

    grid = (B, HW // CHUNK)
    out = pl.pallas_call(
        _vq_body,
        grid=grid,
        in_specs=[
            pl.BlockSpec((1, C, CHUNK), lambda b, t: (b, 0, t)),
            pl.BlockSpec((MEMC, C), lambda b, t: (0, 0)),
            pl.BlockSpec((MEMC, C), lambda b, t: (0, 0)),
        ],
        out_specs=pl.BlockSpec((1, C, CHUNK), lambda b, t: (b, 0, t)),
        out_shape=jax.ShapeDtypeStruct((B, C, HW), x.dtype),
        compiler_params=pltpu.CompilerParams(
            dimension_semantics=("parallel", "arbitrary"),
            fuse_transposed_lhs_in_matmul=True,
        ),
    )(x3, mem_norm, memory)

    return out.reshape(B, C, H, W)


# speedup vs baseline: 2.3365x; 1.1791x over previous
import jax
import jax.numpy as jnp
from jax.experimental import pallas as pl

MEMC = 1024
CHUNK = 4096


def _vq_body(x_ref, mn_ref, mem_ref, o_ref):
    xc = x_ref[0]
    n = jnp.sqrt(jnp.sum(xc * xc, axis=0, keepdims=True))
    xn = xc / jnp.maximum(n, 1e-12)
    sim = jax.lax.dot_general(xn, mn_ref[...], (((0,), (1,)), ((), ())))
    idx = jnp.argmax(sim, axis=1)
    k_iota = jax.lax.broadcasted_iota(jnp.int32, (CHUNK, MEMC), 1)
    oh = (k_iota == idx[:, None]).astype(jnp.float32)
    o_ref[0] = jax.lax.dot_general(mem_ref[...], oh, (((0,), (1,)), ((), ())))


@jax.jit
def kernel(x, memory):
    B, C, H, W = x.shape
    HW = H * W
    x3 = x.reshape(B, C, HW)

    mn = jnp.linalg.norm(memory, axis=1, keepdims=True)
    mem_norm = memory / jnp.maximum(mn, 1e-12)

    grid = (B, HW // CHUNK)
    out = pl.pallas_call(
        _vq_body,
        grid=grid,
        in_specs=[
            pl.BlockSpec((1, C, CHUNK), lambda b, t: (b, 0, t)),
            pl.BlockSpec((MEMC, C), lambda b, t: (0, 0)),
            pl.BlockSpec((MEMC, C), lambda b, t: (0, 0)),
        ],
        out_specs=pl.BlockSpec((1, C, CHUNK), lambda b, t: (b, 0, t)),
        out_shape=jax.ShapeDtypeStruct((B, C, HW), x.dtype),
    )(x3, mem_norm, memory)

    return out.reshape(B, C, H, W)
